# Initial kernel scaffold; baseline (speedup 1.0000x reference)
import functools

import jax
import jax.numpy as jnp
from jax import lax
from jax.experimental import pallas as pl
from jax.experimental.pallas import tpu as pltpu
from jax.experimental.pallas import tpu_sc as plsc

_B_ROWS = 4096
_SEQ = 50
_D = 64
_B = _B_ROWS * _SEQ
_NC = 2
_NS = 16
_NW = _NC * _NS
_BW = _B // _NW
_C = 128
_NCH = _BW // _C

_mesh = plsc.VectorSubcoreMesh(core_axis_name="c", subcore_axis_name="s")


@functools.partial(
    pl.kernel,
    mesh=_mesh,
    out_type=jax.ShapeDtypeStruct((_B, _D), jnp.float32),
    scratch_types=[
        pltpu.VMEM((_NCH, _C), jnp.int32),
        pltpu.VMEM((_C, _D), jnp.float32),
        pltpu.SemaphoreType.DMA,
    ],
)
def _sc_gather(idx_hbm, table_hbm, out_hbm, idx_v, rows_v, sem):
    wid = lax.axis_index("s") * _NC + lax.axis_index("c")
    pltpu.sync_copy(idx_hbm.at[pl.ds(wid * _NCH, _NCH)], idx_v)
    row0 = wid * _BW

    def step(j, carry):
        pltpu.async_copy(table_hbm.at[idx_v.at[j]], rows_v, sem).wait()
        pltpu.sync_copy(rows_v, out_hbm.at[pl.ds(row0 + j * _C, _C)])
        return carry

    lax.fori_loop(0, _NCH, step, 0)


def kernel(inputs, embedding):
    idx = inputs.astype(jnp.int32).reshape(_NW * _NCH, _C)
    out = _sc_gather(idx, embedding)
    return out.reshape(_B_ROWS, _SEQ, _D)


# speedup vs baseline: 4.1023x; 4.1023x over previous
import functools

import jax
import jax.numpy as jnp
from jax import lax
from jax.experimental import pallas as pl
from jax.experimental.pallas import tpu as pltpu
from jax.experimental.pallas import tpu_sc as plsc

_B_ROWS = 4096
_SEQ = 50
_D = 64
_B = _B_ROWS * _SEQ
_NC = 2
_NS = 16
_NW = _NC * _NS
_BW = _B // _NW
_C = 128
_NCH = _BW // _C

_mesh = plsc.VectorSubcoreMesh(core_axis_name="c", subcore_axis_name="s")


@functools.partial(
    pl.kernel,
    mesh=_mesh,
    out_type=jax.ShapeDtypeStruct((_B, _D), jnp.float32),
    scratch_types=[
        pltpu.VMEM((_NCH, _C), jnp.int32),
        pltpu.VMEM((_C, _D), jnp.float32),
        pltpu.SemaphoreType.DMA,
    ],
    compiler_params=pltpu.CompilerParams(use_tc_tiling_on_sc=False),
)
def _sc_gather(idx_hbm, table_hbm, out_hbm, idx_v, rows_v, sem):
    wid = lax.axis_index("s") * _NC + lax.axis_index("c")
    pltpu.sync_copy(idx_hbm.at[wid], idx_v)
    row0 = wid * _BW

    def step(j, carry):
        pltpu.async_copy(table_hbm.at[idx_v.at[j]], rows_v, sem).wait()
        pltpu.sync_copy(rows_v, out_hbm.at[pl.ds(row0 + j * _C, _C)])
        return carry

    lax.fori_loop(0, _NCH, step, 0)


def kernel(inputs, embedding):
    idx = inputs.astype(jnp.int32).reshape(_NW, _NCH, _C)
    out = _sc_gather(idx, embedding)
    return out.reshape(_B_ROWS, _SEQ, _D)


# speedup vs baseline: 4.6737x; 1.1393x over previous
import functools

import jax
import jax.numpy as jnp
from jax import lax
from jax.experimental import pallas as pl
from jax.experimental.pallas import tpu as pltpu
from jax.experimental.pallas import tpu_sc as plsc

_B_ROWS = 4096
_SEQ = 50
_D = 64
_B = _B_ROWS * _SEQ
_NC = 2
_NS = 16
_NW = _NC * _NS
_BW = _B // _NW
_C = 128
_NCH = _BW // _C
_NBUF = 5
_NT = _NCH // _NBUF

_mesh = plsc.VectorSubcoreMesh(core_axis_name="c", subcore_axis_name="s")


@functools.partial(
    pl.kernel,
    mesh=_mesh,
    out_type=jax.ShapeDtypeStruct((_B, _D), jnp.float32),
    scratch_types=[
        pltpu.VMEM((_BW,), jnp.int32),
        pltpu.VMEM((_NBUF, _C, _D), jnp.float32),
        pltpu.SemaphoreType.DMA((_NBUF,)),
        pltpu.SemaphoreType.DMA((_NBUF,)),
    ],
    compiler_params=pltpu.CompilerParams(use_tc_tiling_on_sc=False),
)
def _sc_gather(idx_hbm, table_hbm, out_hbm, idx_v, rows_v, gsem, wsem):
    wid = lax.axis_index("s") * _NC + lax.axis_index("c")
    pltpu.sync_copy(idx_hbm.at[pl.ds(wid * _BW, _BW)], idx_v)
    row0 = wid * _BW

    def gather_start(b, j):
        pltpu.make_async_copy(
            table_hbm.at[idx_v.at[pl.ds(j * _C, _C)]], rows_v.at[b], gsem.at[b]
        ).start()

    def gather_wait(b, j):
        pltpu.make_async_copy(
            table_hbm.at[idx_v.at[pl.ds(j * _C, _C)]], rows_v.at[b], gsem.at[b]
        ).wait()

    def write_start(b, j):
        pltpu.make_async_copy(
            rows_v.at[b], out_hbm.at[pl.ds(row0 + j * _C, _C)], wsem.at[b]
        ).start()

    def write_wait(b, j):
        pltpu.make_async_copy(
            rows_v.at[b], out_hbm.at[pl.ds(row0 + j * _C, _C)], wsem.at[b]
        ).wait()

    for b in range(_NBUF):
        gather_start(b, b)

    def step(t, carry):
        for b in range(_NBUF):
            j = t * _NBUF + b
            gather_wait(b, j)
            write_start(b, j)
            jn = j + _NBUF

            @pl.when(jn < _NCH)
            def _():
                write_wait(b, j)
                gather_start(b, jn)

        return carry

    lax.fori_loop(0, _NT, step, 0)

    for b in range(_NBUF):
        write_wait(b, (_NT - 1) * _NBUF + b)


def kernel(inputs, embedding):
    idx = inputs.astype(jnp.int32).reshape(_B)
    out = _sc_gather(idx, embedding)
    return out.reshape(_B_ROWS, _SEQ, _D)
